# baseline (speedup 1.0000x reference)
import functools

import jax
import jax.numpy as jnp
from jax import lax
from jax.experimental import pallas as pl
from jax.experimental.pallas import tpu as pltpu
from jax.experimental.pallas import tpu_sc as plsc

B = 4096
EMB = 32

_NC, _NS = 2, 16
_NW = _NC * _NS
_BPW = B // _NW


def _gather_body(uid_hbm, sid_hbm, ut_hbm, st_hbm, uout_hbm, sout_hbm,
                 uidx_v, urows_v, sidx_v, srows_v, usem, ssem):
    wid = lax.axis_index("s") * _NC + lax.axis_index("c")
    base = wid * _BPW
    pltpu.sync_copy(uid_hbm.at[pl.ds(base, _BPW)], uidx_v)
    pltpu.sync_copy(sid_hbm.at[pl.ds(base, _BPW)], sidx_v)
    ucopy = pltpu.async_copy(ut_hbm.at[uidx_v], urows_v, usem)
    scopy = pltpu.async_copy(st_hbm.at[sidx_v], srows_v, ssem)
    ucopy.wait()
    scopy.wait()
    pltpu.sync_copy(urows_v, uout_hbm.at[pl.ds(base, _BPW)])
    pltpu.sync_copy(srows_v, sout_hbm.at[pl.ds(base, _BPW)])


@functools.cache
def _sc_gather():
    return pl.kernel(
        _gather_body,
        mesh=plsc.VectorSubcoreMesh(core_axis_name="c", subcore_axis_name="s",
                                    num_cores=_NC, num_subcores=_NS),
        out_type=[
            jax.ShapeDtypeStruct((B, EMB), jnp.float32),
            jax.ShapeDtypeStruct((B, EMB), jnp.float32),
        ],
        scratch_types=[
            pltpu.VMEM((_BPW,), jnp.int32),
            pltpu.VMEM((_BPW, EMB), jnp.float32),
            pltpu.VMEM((_BPW,), jnp.int32),
            pltpu.VMEM((_BPW, EMB), jnp.float32),
            pltpu.SemaphoreType.DMA,
            pltpu.SemaphoreType.DMA,
        ],
        compiler_params=pltpu.CompilerParams(use_tc_tiling_on_sc=False),
    )


def _mlp_body(u_ref, s_ref, w1u_ref, w1s_ref, b1_ref, w2_ref, b2_ref,
              w3_ref, b3_ref, out_ref):
    h = jnp.dot(u_ref[...], w1u_ref[...], preferred_element_type=jnp.float32)
    h += jnp.dot(s_ref[...], w1s_ref[...], preferred_element_type=jnp.float32)
    h = jnp.maximum(h + b1_ref[...], 0.0)
    h = jnp.maximum(
        jnp.dot(h, w2_ref[...], preferred_element_type=jnp.float32)
        + b2_ref[...], 0.0)
    out_ref[...] = (
        jnp.dot(h, w3_ref[...], preferred_element_type=jnp.float32)
        + b3_ref[...])


def _mlp(u_emb, s_emb, W1u, W1s, b1, W2, b2, W3, b3):
    nb = 4
    rows = B // nb
    return pl.pallas_call(
        _mlp_body,
        grid=(nb,),
        in_specs=[
            pl.BlockSpec((rows, EMB), lambda i: (i, 0)),
            pl.BlockSpec((rows, EMB), lambda i: (i, 0)),
            pl.BlockSpec((EMB, 256), lambda i: (0, 0)),
            pl.BlockSpec((EMB, 256), lambda i: (0, 0)),
            pl.BlockSpec((1, 256), lambda i: (0, 0)),
            pl.BlockSpec((256, 64), lambda i: (0, 0)),
            pl.BlockSpec((1, 64), lambda i: (0, 0)),
            pl.BlockSpec((64, 1), lambda i: (0, 0)),
            pl.BlockSpec((1, 1), lambda i: (0, 0)),
        ],
        out_specs=pl.BlockSpec((rows, 1), lambda i: (i, 0)),
        out_shape=jax.ShapeDtypeStruct((B, 1), jnp.float32),
    )(u_emb, s_emb, W1u, W1s, b1, W2, b2, W3, b3)


@jax.jit
def kernel(user_id, song_id, user_table, song_table, W1, b1, W2, b2, W3, b3):
    u_emb, s_emb = _sc_gather()(user_id.astype(jnp.int32),
                                song_id.astype(jnp.int32),
                                user_table, song_table)
    return _mlp(u_emb, s_emb, W1[:EMB], W1[EMB:], b1.reshape(1, 256),
                W2, b2.reshape(1, 64), W3, b3.reshape(1, 1))
